# baseline (speedup 1.0000x reference)
import functools

import jax
import jax.numpy as jnp
import numpy as np
from jax import lax
from jax.experimental import pallas as pl
from jax.experimental.pallas import tpu as pltpu
from jax.experimental.pallas import tpu_sc as plsc

B, S, D = 2, 2048, 1024
E = 8
H = D * 4
N = B * S

TG = 512
NTG = N // TG
T = 512
NT = N // T
HB = 1024
NH = H // HB
W = NT + E - 1


def _gelu(x):
    return 0.5 * x * (1.0 + lax.erf(x * np.float32(1.0 / np.sqrt(2.0))))


def _dot_bf16(a, b):
    return jnp.dot(a, b, preferred_element_type=jnp.float32)


def _gate_body(x_ref, gw_ref, gb_ref, idx_ref, sc_ref, pw_ref, ps_ref,
               cnt_ref, loss_ref):
    pid = pl.program_id(0)
    logits = jnp.dot(x_ref[...], gw_ref[...],
                     preferred_element_type=jnp.float32) + gb_ref[...]
    m = jnp.max(logits, axis=1, keepdims=True)
    ex = jnp.exp(logits - m)
    s = jnp.sum(ex, axis=1, keepdims=True)
    probs = ex / s
    pmax = jnp.max(probs, axis=1, keepdims=True)
    lanes = lax.broadcasted_iota(jnp.int32, (TG, E), 1)
    amax = jnp.min(jnp.where(probs == pmax, lanes, E), axis=1, keepdims=True)
    idx_ref[...] = amax
    sc_ref[...] = pmax

    @pl.when(pid == 0)
    def _():
        ps_ref[...] = jnp.zeros_like(ps_ref)
        cnt_ref[...] = jnp.zeros_like(cnt_ref)
        loss_ref[...] = jnp.zeros_like(loss_ref)

    onehot = (lanes == amax).astype(jnp.float32)
    ri = lax.broadcasted_iota(jnp.int32, (TG, TG), 0)
    ci = lax.broadcasted_iota(jnp.int32, (TG, TG), 1)
    tri = (ri > ci).astype(jnp.float32)
    within = jnp.dot(tri, onehot, preferred_element_type=jnp.float32)
    base = cnt_ref[...]
    pw_ref[...] = jnp.sum((within + base) * onehot, axis=1,
                          keepdims=True).astype(jnp.int32)

    ps_ref[...] += jnp.sum(probs, axis=0, keepdims=True)
    cnt_ref[...] += jnp.sum(onehot, axis=0, keepdims=True)

    @pl.when(pid == pl.num_programs(0) - 1)
    def _():
        loss_ref[...] = (np.float32(E) / np.float32(N * N)
                         * jnp.sum(ps_ref[...] * cnt_ref[...])).reshape(1, 1)


def _gating(x2d, gate_w, gate_b2d):
    return pl.pallas_call(
        _gate_body,
        grid=(NTG,),
        in_specs=[
            pl.BlockSpec((TG, D), lambda i: (i, 0)),
            pl.BlockSpec((D, E), lambda i: (0, 0)),
            pl.BlockSpec((1, E), lambda i: (0, 0)),
        ],
        out_specs=[
            pl.BlockSpec((TG, 1), lambda i: (i, 0)),
            pl.BlockSpec((TG, 1), lambda i: (i, 0)),
            pl.BlockSpec((TG, 1), lambda i: (i, 0)),
            pl.BlockSpec((1, E), lambda i: (0, 0)),
            pl.BlockSpec((1, E), lambda i: (0, 0)),
            pl.BlockSpec((1, 1), lambda i: (0, 0)),
        ],
        out_shape=[
            jax.ShapeDtypeStruct((N, 1), jnp.int32),
            jax.ShapeDtypeStruct((N, 1), jnp.float32),
            jax.ShapeDtypeStruct((N, 1), jnp.int32),
            jax.ShapeDtypeStruct((1, E), jnp.float32),
            jax.ShapeDtypeStruct((1, E), jnp.float32),
            jax.ShapeDtypeStruct((1, 1), jnp.float32),
        ],
        compiler_params=pltpu.CompilerParams(
            dimension_semantics=("arbitrary",)),
    )(x2d, gate_w, gate_b2d)


def _make_sc_gather():
    nc, ns = 2, 16
    nw = nc * ns
    rows_per_w = N // nw
    chunk = 64
    nchunk = rows_per_w // chunk
    mesh = plsc.VectorSubcoreMesh(core_axis_name="c", subcore_axis_name="s",
                                  num_cores=nc, num_subcores=ns)

    @functools.partial(
        pl.kernel,
        mesh=mesh,
        out_type=jax.ShapeDtypeStruct((N, D), jnp.float32),
        scratch_types=[
            pltpu.VMEM((chunk,), jnp.int32),
            pltpu.VMEM((chunk, D), jnp.float32),
            pltpu.SemaphoreType.DMA,
        ],
    )
    def sc_gather(table_hbm, idx_hbm, out_hbm, idx_v, rows_v, sem):
        wid = lax.axis_index("s") * nc + lax.axis_index("c")
        base = wid * rows_per_w
        for c in range(nchunk):
            off = base + c * chunk
            pltpu.sync_copy(idx_hbm.at[pl.ds(off, chunk)], idx_v)
            pltpu.async_copy(table_hbm.at[idx_v], rows_v, sem).wait()
            pltpu.sync_copy(rows_v, out_hbm.at[pl.ds(off, chunk)])

    return sc_gather


def _make_sc_scatter():
    nc, ns = 2, 16
    nw = nc * ns
    rows_per_w = N // nw
    chunk = 64
    nchunk = rows_per_w // chunk
    mesh = plsc.VectorSubcoreMesh(core_axis_name="c", subcore_axis_name="s",
                                  num_cores=nc, num_subcores=ns)

    @functools.partial(
        pl.kernel,
        mesh=mesh,
        out_type=jax.ShapeDtypeStruct((N, D), jnp.float32),
        scratch_types=[
            pltpu.VMEM((chunk,), jnp.int32),
            pltpu.VMEM((chunk, D), jnp.float32),
            pltpu.SemaphoreType.DMA,
        ],
    )
    def sc_scatter(table_hbm, idx_hbm, out_hbm, idx_v, rows_v, sem):
        wid = lax.axis_index("s") * nc + lax.axis_index("c")
        base = wid * rows_per_w
        for c in range(nchunk):
            off = base + c * chunk
            pltpu.sync_copy(idx_hbm.at[pl.ds(off, chunk)], idx_v)
            pltpu.sync_copy(table_hbm.at[pl.ds(off, chunk)], rows_v)
            pltpu.async_copy(rows_v, out_hbm.at[idx_v], sem).wait()

    return sc_scatter


_SC_SCATTER_CACHE = []


def _sc_scatter(table, idx):
    if not _SC_SCATTER_CACHE:
        _SC_SCATTER_CACHE.append(_make_sc_scatter())
    return _SC_SCATTER_CACHE[0](table, idx)


_SC_GATHER_CACHE = []


def _sc_gather(table, idx):
    if not _SC_GATHER_CACHE:
        _SC_GATHER_CACHE.append(_make_sc_gather())
    return _SC_GATHER_CACHE[0](table, idx)


def _mlp_body(tid_ref, eid_ref, fst_ref, nw_ref, xs_ref, w1_ref, w2_ref,
              b1_ref, b2_ref, sw_ref, out_ref, acc_ref, w1c_ref, w2c_ref):
    h = pl.program_id(0)
    w = pl.program_id(1)

    @pl.when(nw_ref[w] == 1)
    def _():
        w1c_ref[...] = w1_ref[0].astype(jnp.bfloat16)
        w2c_ref[...] = w2_ref[0].astype(jnp.bfloat16)

    xt = xs_ref[...]
    hpre0 = _dot_bf16(xt, w1c_ref[:, :HB // 2]) + b1_ref[0, :, :HB // 2]
    g0 = _gelu(hpre0).astype(jnp.bfloat16)
    hpre1 = _dot_bf16(xt, w1c_ref[:, HB // 2:]) + b1_ref[0, :, HB // 2:]
    g1 = _gelu(hpre1).astype(jnp.bfloat16)
    part = (_dot_bf16(g0, w2c_ref[:HB // 2, :])
            + _dot_bf16(g1, w2c_ref[HB // 2:, :]))
    swc = sw_ref[0]
    row = tid_ref[w] * T

    @pl.when((h == 0) & (fst_ref[w] == 1))
    def _():
        acc_ref[pl.ds(row, T), :] = swc * (part + b2_ref[0])

    @pl.when((h == 0) & (fst_ref[w] == 0))
    def _():
        acc_ref[pl.ds(row, T), :] += swc * (part + b2_ref[0])

    @pl.when(h != 0)
    def _():
        acc_ref[pl.ds(row, T), :] += swc * part

    @pl.when(h == NH - 1)
    def _():
        out_ref[...] = acc_ref[pl.ds(row, T), :]


def _grouped_mlp(tile_ids, expert_ids, first, new_w, xs, w1, w2, b1_w, b2_w,
                 sw):
    grid_spec = pltpu.PrefetchScalarGridSpec(
        num_scalar_prefetch=4,
        grid=(NH, W),
        in_specs=[
            pl.BlockSpec((T, D),
                         lambda h, w, tid, eid, fst, nw: (tid[w], 0)),
            pl.BlockSpec((1, D, HB),
                         lambda h, w, tid, eid, fst, nw: (eid[w], 0, h)),
            pl.BlockSpec((1, HB, D),
                         lambda h, w, tid, eid, fst, nw: (eid[w], h, 0)),
            pl.BlockSpec((1, 1, HB),
                         lambda h, w, tid, eid, fst, nw: (w, 0, h)),
            pl.BlockSpec((1, 1, D),
                         lambda h, w, tid, eid, fst, nw: (w, 0, 0)),
            pl.BlockSpec((1, T, 1),
                         lambda h, w, tid, eid, fst, nw: (w, 0, 0)),
        ],
        out_specs=pl.BlockSpec(
            (T, D),
            lambda h, w, tid, eid, fst, nw:
                (jnp.where(h == NH - 1, tid[w], 0), 0)),
        scratch_shapes=[
            pltpu.VMEM((N, D), jnp.float32),
            pltpu.VMEM((D, HB), jnp.bfloat16),
            pltpu.VMEM((HB, D), jnp.bfloat16),
        ],
    )
    return pl.pallas_call(
        _mlp_body,
        grid_spec=grid_spec,
        out_shape=jax.ShapeDtypeStruct((N, D), jnp.float32),
        compiler_params=pltpu.CompilerParams(
            dimension_semantics=("arbitrary", "arbitrary")),
    )(tile_ids, expert_ids, first, new_w, xs, w1, w2, b1_w, b2_w, sw)


def _work_items(counts):
    ends = jnp.cumsum(counts)
    starts = ends - counts
    t_lo = starts // T
    t_hi = (ends + T - 1) // T
    items_per = jnp.where(counts > 0, t_hi - t_lo, 0)
    item_cum = jnp.cumsum(items_per)
    total = item_cum[-1]
    j = jnp.arange(W, dtype=jnp.int32)
    e_j = jnp.searchsorted(item_cum, j, side="right").astype(jnp.int32)
    e_j = jnp.minimum(e_j, E - 1)
    valid = j < total
    off_e = item_cum[e_j] - items_per[e_j]
    tile_j = t_lo[e_j] + (j - off_e)
    tile_j = jnp.where(valid, tile_j, NT - 1).astype(jnp.int32)
    e_j = jnp.where(valid, e_j, E - 1).astype(jnp.int32)
    glo = jnp.maximum(starts[e_j], tile_j * T)
    ghi = jnp.minimum(ends[e_j], (tile_j + 1) * T)
    lo = jnp.where(valid, glo - tile_j * T, 0).astype(jnp.int32)
    hi = jnp.where(valid, ghi - tile_j * T, 0).astype(jnp.int32)
    prev = jnp.concatenate([jnp.array([-1], jnp.int32), tile_j[:-1]])
    first = (valid & (tile_j != prev)).astype(jnp.int32)
    prev_e = jnp.concatenate([jnp.array([-1], jnp.int32), e_j[:-1]])
    new_w = (e_j != prev_e).astype(jnp.int32)
    return tile_j, e_j, lo, hi, first, new_w


def kernel(x, gate_w, gate_b, w1, b1, w2, b2):
    x2d = x.reshape(N, D)

    idx_c, score_c, pw_c, _ps, cnt, loss = _gating(x2d, gate_w,
                                                   gate_b.reshape(1, E))
    idx = idx_c.reshape(N)
    score = score_c.reshape(N)
    loss_out = loss.reshape(())

    counts = cnt.reshape(E).astype(jnp.int32)
    starts = jnp.cumsum(counts) - counts
    pos = starts[idx] + pw_c.reshape(N)
    tile_j, e_j, lo, hi, first, new_w = _work_items(counts)

    r = jnp.arange(T, dtype=jnp.int32)
    inrange = (r[None, :] >= lo[:, None]) & (r[None, :] < hi[:, None])
    sw = inrange.astype(jnp.float32).reshape(W, T, 1)
    b1_w = b1[e_j].reshape(W, 1, H)
    b2_w = b2[e_j].reshape(W, 1, D)

    xs = _sc_scatter(x2d, pos)
    ys = _grouped_mlp(tile_j, e_j, first, new_w, xs.astype(jnp.bfloat16),
                      w1, w2, b1_w, b2_w, sw)
    out2d = _sc_gather(ys, pos)

    return (out2d * score[:, None]).reshape(B, S, D), loss_out


# speedup vs baseline: 1.0578x; 1.0578x over previous
import functools

import jax
import jax.numpy as jnp
import numpy as np
from jax import lax
from jax.experimental import pallas as pl
from jax.experimental.pallas import tpu as pltpu
from jax.experimental.pallas import tpu_sc as plsc

B, S, D = 2, 2048, 1024
E = 8
H = D * 4
N = B * S

TG = 512
NTG = N // TG
T = 512
NT = N // T
HB = 1024
NH = H // HB
W = NT + E - 1


def _gelu(x):
    return 0.5 * x * (1.0 + lax.erf(x * np.float32(1.0 / np.sqrt(2.0))))


def _dot_bf16(a, b):
    return jnp.dot(a, b, preferred_element_type=jnp.float32)


def _gate_body(x_ref, gw_ref, gb_ref, idx_ref, sc_ref, pw_ref, ps_ref,
               cnt_ref, loss_ref):
    pid = pl.program_id(0)
    logits = jnp.dot(x_ref[...], gw_ref[...],
                     preferred_element_type=jnp.float32) + gb_ref[...]
    m = jnp.max(logits, axis=1, keepdims=True)
    ex = jnp.exp(logits - m)
    s = jnp.sum(ex, axis=1, keepdims=True)
    probs = ex / s
    pmax = jnp.max(probs, axis=1, keepdims=True)
    lanes = lax.broadcasted_iota(jnp.int32, (TG, E), 1)
    amax = jnp.min(jnp.where(probs == pmax, lanes, E), axis=1, keepdims=True)
    idx_ref[...] = amax
    sc_ref[...] = pmax

    @pl.when(pid == 0)
    def _():
        ps_ref[...] = jnp.zeros_like(ps_ref)
        cnt_ref[...] = jnp.zeros_like(cnt_ref)
        loss_ref[...] = jnp.zeros_like(loss_ref)

    onehot = (lanes == amax).astype(jnp.float32)
    ri = lax.broadcasted_iota(jnp.int32, (TG, TG), 0)
    ci = lax.broadcasted_iota(jnp.int32, (TG, TG), 1)
    tri = (ri > ci).astype(jnp.float32)
    within = jnp.dot(tri, onehot, preferred_element_type=jnp.float32)
    base = cnt_ref[...]
    pw_ref[...] = jnp.sum((within + base) * onehot, axis=1,
                          keepdims=True).astype(jnp.int32)

    ps_ref[...] += jnp.sum(probs, axis=0, keepdims=True)
    cnt_ref[...] += jnp.sum(onehot, axis=0, keepdims=True)

    @pl.when(pid == pl.num_programs(0) - 1)
    def _():
        loss_ref[...] = (np.float32(E) / np.float32(N * N)
                         * jnp.sum(ps_ref[...] * cnt_ref[...])).reshape(1, 1)


def _gating(x2d, gate_w, gate_b2d):
    return pl.pallas_call(
        _gate_body,
        grid=(NTG,),
        in_specs=[
            pl.BlockSpec((TG, D), lambda i: (i, 0)),
            pl.BlockSpec((D, E), lambda i: (0, 0)),
            pl.BlockSpec((1, E), lambda i: (0, 0)),
        ],
        out_specs=[
            pl.BlockSpec((TG, 1), lambda i: (i, 0)),
            pl.BlockSpec((TG, 1), lambda i: (i, 0)),
            pl.BlockSpec((TG, 1), lambda i: (i, 0)),
            pl.BlockSpec((1, E), lambda i: (0, 0)),
            pl.BlockSpec((1, E), lambda i: (0, 0)),
            pl.BlockSpec((1, 1), lambda i: (0, 0)),
        ],
        out_shape=[
            jax.ShapeDtypeStruct((N, 1), jnp.int32),
            jax.ShapeDtypeStruct((N, 1), jnp.float32),
            jax.ShapeDtypeStruct((N, 1), jnp.int32),
            jax.ShapeDtypeStruct((1, E), jnp.float32),
            jax.ShapeDtypeStruct((1, E), jnp.float32),
            jax.ShapeDtypeStruct((1, 1), jnp.float32),
        ],
        compiler_params=pltpu.CompilerParams(
            dimension_semantics=("arbitrary",)),
    )(x2d, gate_w, gate_b2d)


def _make_sc_gather():
    nc, ns = 2, 16
    nw = nc * ns
    rows_per_w = N // nw
    chunk = 64
    nchunk = rows_per_w // chunk
    mesh = plsc.VectorSubcoreMesh(core_axis_name="c", subcore_axis_name="s",
                                  num_cores=nc, num_subcores=ns)

    @functools.partial(
        pl.kernel,
        mesh=mesh,
        out_type=jax.ShapeDtypeStruct((N, D), jnp.float32),
        scratch_types=[
            pltpu.VMEM((chunk,), jnp.int32),
            pltpu.VMEM((chunk, D), jnp.float32),
            pltpu.SemaphoreType.DMA,
        ],
    )
    def sc_gather(table_hbm, idx_hbm, out_hbm, idx_v, rows_v, sem):
        wid = lax.axis_index("s") * nc + lax.axis_index("c")
        base = wid * rows_per_w
        for c in range(nchunk):
            off = base + c * chunk
            pltpu.sync_copy(idx_hbm.at[pl.ds(off, chunk)], idx_v)
            pltpu.async_copy(table_hbm.at[idx_v], rows_v, sem).wait()
            pltpu.sync_copy(rows_v, out_hbm.at[pl.ds(off, chunk)])

    return sc_gather


def _make_sc_scatter():
    nc, ns = 2, 16
    nw = nc * ns
    rows_per_w = N // nw
    chunk = 64
    nchunk = rows_per_w // chunk
    mesh = plsc.VectorSubcoreMesh(core_axis_name="c", subcore_axis_name="s",
                                  num_cores=nc, num_subcores=ns)

    @functools.partial(
        pl.kernel,
        mesh=mesh,
        out_type=jax.ShapeDtypeStruct((N, D), jnp.float32),
        scratch_types=[
            pltpu.VMEM((chunk,), jnp.int32),
            pltpu.VMEM((chunk, D), jnp.float32),
            pltpu.SemaphoreType.DMA,
        ],
    )
    def sc_scatter(table_hbm, idx_hbm, out_hbm, idx_v, rows_v, sem):
        wid = lax.axis_index("s") * nc + lax.axis_index("c")
        base = wid * rows_per_w
        for c in range(nchunk):
            off = base + c * chunk
            pltpu.sync_copy(idx_hbm.at[pl.ds(off, chunk)], idx_v)
            pltpu.sync_copy(table_hbm.at[pl.ds(off, chunk)], rows_v)
            pltpu.async_copy(rows_v, out_hbm.at[idx_v], sem).wait()

    return sc_scatter


_SC_SCATTER_CACHE = []


def _sc_scatter(table, idx):
    if not _SC_SCATTER_CACHE:
        _SC_SCATTER_CACHE.append(_make_sc_scatter())
    return _SC_SCATTER_CACHE[0](table, idx)


_SC_GATHER_CACHE = []


def _sc_gather(table, idx):
    if not _SC_GATHER_CACHE:
        _SC_GATHER_CACHE.append(_make_sc_gather())
    return _SC_GATHER_CACHE[0](table, idx)


def _mlp_body(tid_ref, eid_ref, fst_ref, xs_ref, w1_ref, w2_ref, b1_ref,
              b2_ref, sw_ref, out_ref, acc_ref):
    w = pl.program_id(0)
    h = pl.program_id(1)
    hpre = _dot_bf16(xs_ref[...], w1_ref[0].astype(jnp.bfloat16)) + b1_ref[0]
    g = _gelu(hpre)
    part = _dot_bf16(g.astype(jnp.bfloat16), w2_ref[0].astype(jnp.bfloat16))

    @pl.when(h == 0)
    def _():
        acc_ref[...] = part

    @pl.when((h != 0) & (h != NH - 1))
    def _():
        acc_ref[...] += part

    @pl.when(h == NH - 1)
    def _():
        swc = sw_ref[0]
        contrib = swc * (acc_ref[...] + part + b2_ref[0])

        @pl.when(fst_ref[w] == 1)
        def _():
            out_ref[...] = contrib

        @pl.when(fst_ref[w] == 0)
        def _():
            out_ref[...] += contrib


def _grouped_mlp(tile_ids, expert_ids, first, xs, w1, w2, b1_w, b2_w, sw):
    grid_spec = pltpu.PrefetchScalarGridSpec(
        num_scalar_prefetch=3,
        grid=(W, NH),
        in_specs=[
            pl.BlockSpec((T, D), lambda w, h, tid, eid, fst: (tid[w], 0)),
            pl.BlockSpec((1, D, HB), lambda w, h, tid, eid, fst: (eid[w], 0, h)),
            pl.BlockSpec((1, HB, D), lambda w, h, tid, eid, fst: (eid[w], h, 0)),
            pl.BlockSpec((1, 1, HB), lambda w, h, tid, eid, fst: (w, 0, h)),
            pl.BlockSpec((1, 1, D), lambda w, h, tid, eid, fst: (w, 0, 0)),
            pl.BlockSpec((1, T, 1), lambda w, h, tid, eid, fst: (w, 0, 0)),
        ],
        out_specs=pl.BlockSpec((T, D),
                               lambda w, h, tid, eid, fst: (tid[w], 0)),
        scratch_shapes=[pltpu.VMEM((T, D), jnp.float32)],
    )
    return pl.pallas_call(
        _mlp_body,
        grid_spec=grid_spec,
        out_shape=jax.ShapeDtypeStruct((N, D), jnp.float32),
        compiler_params=pltpu.CompilerParams(
            dimension_semantics=("arbitrary", "arbitrary")),
    )(tile_ids, expert_ids, first, xs, w1, w2, b1_w, b2_w, sw)


def _work_items(counts):
    ends = jnp.cumsum(counts)
    starts = ends - counts
    t_lo = starts // T
    t_hi = (ends + T - 1) // T
    items_per = jnp.where(counts > 0, t_hi - t_lo, 0)
    item_cum = jnp.cumsum(items_per)
    total = item_cum[-1]
    j = jnp.arange(W, dtype=jnp.int32)
    e_j = jnp.searchsorted(item_cum, j, side="right").astype(jnp.int32)
    e_j = jnp.minimum(e_j, E - 1)
    valid = j < total
    off_e = item_cum[e_j] - items_per[e_j]
    tile_j = t_lo[e_j] + (j - off_e)
    tile_j = jnp.where(valid, tile_j, NT - 1).astype(jnp.int32)
    e_j = jnp.where(valid, e_j, E - 1).astype(jnp.int32)
    glo = jnp.maximum(starts[e_j], tile_j * T)
    ghi = jnp.minimum(ends[e_j], (tile_j + 1) * T)
    lo = jnp.where(valid, glo - tile_j * T, 0).astype(jnp.int32)
    hi = jnp.where(valid, ghi - tile_j * T, 0).astype(jnp.int32)
    prev = jnp.concatenate([jnp.array([-1], jnp.int32), tile_j[:-1]])
    first = (valid & (tile_j != prev)).astype(jnp.int32)
    prev_e = jnp.concatenate([jnp.array([-1], jnp.int32), e_j[:-1]])
    new_w = (e_j != prev_e).astype(jnp.int32)
    return tile_j, e_j, lo, hi, first, new_w


def kernel(x, gate_w, gate_b, w1, b1, w2, b2):
    x2d = x.reshape(N, D)

    idx_c, score_c, pw_c, _ps, cnt, loss = _gating(x2d, gate_w,
                                                   gate_b.reshape(1, E))
    idx = idx_c.reshape(N)
    score = score_c.reshape(N)
    loss_out = loss.reshape(())

    counts = cnt.reshape(E).astype(jnp.int32)
    starts = jnp.cumsum(counts) - counts
    pos = starts[idx] + pw_c.reshape(N)
    tile_j, e_j, lo, hi, first, new_w = _work_items(counts)

    r = jnp.arange(T, dtype=jnp.int32)
    inrange = (r[None, :] >= lo[:, None]) & (r[None, :] < hi[:, None])
    sw = inrange.astype(jnp.float32).reshape(W, T, 1)
    b1_w = b1[e_j].reshape(W, 1, H)
    b2_w = b2[e_j].reshape(W, 1, D)

    xs = _sc_scatter(x2d, pos)
    ys = _grouped_mlp(tile_j, e_j, first, xs.astype(jnp.bfloat16),
                      w1, w2, b1_w, b2_w, sw)
    out2d = _sc_gather(ys, pos)

    return (out2d * score[:, None]).reshape(B, S, D), loss_out
